# Initial kernel scaffold; baseline (speedup 1.0000x reference)
import jax
import jax.numpy as jnp
from jax.experimental import pallas as pl


def kernel(feats):
    raise NotImplementedError("write your pallas kernel here")


# speedup vs baseline: 1.4616x; 1.4616x over previous
import jax
import jax.numpy as jnp
from jax.experimental import pallas as pl
from jax.experimental.pallas import tpu as pltpu

_BLOCK_ROWS = 2048
_FEAT = 512


def _l2norm_body(x_ref, o_ref):
    x = x_ref[...]
    ss = jnp.sum(x * x, axis=-1, keepdims=True)
    o_ref[...] = x * jax.lax.rsqrt(ss)


def kernel(feats):
    b, t, f = feats.shape
    rows = b * t
    x2d = feats.reshape(rows, f)
    grid = (rows // _BLOCK_ROWS,)
    out = pl.pallas_call(
        _l2norm_body,
        out_shape=jax.ShapeDtypeStruct((rows, f), feats.dtype),
        grid=grid,
        in_specs=[pl.BlockSpec((_BLOCK_ROWS, _FEAT), lambda i: (i, 0))],
        out_specs=pl.BlockSpec((_BLOCK_ROWS, _FEAT), lambda i: (i, 0)),
        compiler_params=pltpu.CompilerParams(
            dimension_semantics=("arbitrary",),
        ),
        name="l2_normalize",
    )(x2d)
    return out.reshape(b, t, f)


# speedup vs baseline: 1.4750x; 1.0092x over previous
import jax
import jax.numpy as jnp
from jax.experimental import pallas as pl
from jax.experimental.pallas import tpu as pltpu

_BLOCK_ROWS = 4096
_FEAT = 512


def _l2norm_body(x_ref, o_ref):
    x = x_ref[...]
    ss = jnp.sum(x * x, axis=-1, keepdims=True)
    o_ref[...] = x * jax.lax.rsqrt(ss)


def kernel(feats):
    b, t, f = feats.shape
    rows = b * t
    x2d = feats.reshape(rows, f)
    grid = (rows // _BLOCK_ROWS,)
    out = pl.pallas_call(
        _l2norm_body,
        out_shape=jax.ShapeDtypeStruct((rows, f), feats.dtype),
        grid=grid,
        in_specs=[pl.BlockSpec((_BLOCK_ROWS, _FEAT), lambda i: (i, 0))],
        out_specs=pl.BlockSpec((_BLOCK_ROWS, _FEAT), lambda i: (i, 0)),
        compiler_params=pltpu.CompilerParams(
            dimension_semantics=("arbitrary",),
        ),
        name="l2_normalize",
    )(x2d)
    return out.reshape(b, t, f)
